# baseline (speedup 1.0000x reference)
import jax
import jax.numpy as jnp
from jax import lax
from jax.experimental import pallas as pl
from jax.experimental.pallas import tpu as pltpu

MASK_LEN = 4915
B, T, D = 4, 8192, 1024
BT = 1024
NTB = T // BT
NSEM = 8


def _body(start_ref, x_hbm, mt_ref, out_hbm, mask_ref, tile, sems):
    L = MASK_LEN

    tile[...] = jnp.broadcast_to(mt_ref[...][None], (1, BT, D))
    pos_row = lax.broadcasted_iota(jnp.int32, (1, T), 1)
    for b in range(B):
        s = start_ref[b]
        mask_ref[b : b + 1, :] = (pos_row >= s) & (pos_row < s + L)

    fills = []
    q = 0
    for b in range(B):
        s = start_ref[b]
        sb = s // BT
        eb = (s + L - 1) // BT
        for t in range(1, NTB - 1):
            d = pltpu.make_async_copy(
                tile.at[pl.ds(0, 1), pl.ds(0, BT)],
                out_hbm.at[pl.ds(b, 1), pl.ds(t * BT, BT)],
                sems.at[q % NSEM])
            fills.append(((t > sb) & (t < eb), d))
            q += 1
    for cond, d in fills:
        pl.when(cond)(d.start)

    def blk_of(b):
        s = start_ref[b]
        return s // BT, (s + L - 1) // BT

    def skip_index(b, t):
        sb, eb = blk_of(b)
        interior = (t > sb) & (t < eb)
        return (b, jnp.where(interior, sb, t), 0)

    def copy_body(idx, x_blk, out_blk):
        b, t = idx
        s = start_ref[b]
        sb = s // BT
        eb = (s + L - 1) // BT
        interior = (t > sb) & (t < eb)

        @pl.when(jnp.logical_not(interior))
        def _():
            pos = lax.broadcasted_iota(jnp.int32, (BT, 1), 0) + t * BT
            m = (pos >= s) & (pos < s + L)
            out_blk[0] = jnp.where(m, mt_ref[...], x_blk[0])

    pltpu.emit_pipeline(
        copy_body,
        grid=(B, NTB),
        in_specs=[pl.BlockSpec((1, BT, D), skip_index)],
        out_specs=[pl.BlockSpec((1, BT, D), skip_index)],
        _explicit_indices=True,
    )(x_hbm, out_hbm)

    for cond, d in fills:
        pl.when(cond)(d.wait)


def kernel(x, start_idx, mask_token):
    start_idx = start_idx.astype(jnp.int32)
    x_masked, mask = pl.pallas_call(
        _body,
        in_specs=[
            pl.BlockSpec(memory_space=pltpu.MemorySpace.SMEM),
            pl.BlockSpec(memory_space=pl.ANY),
            pl.BlockSpec(memory_space=pltpu.MemorySpace.VMEM),
        ],
        out_specs=[
            pl.BlockSpec(memory_space=pl.ANY),
            pl.BlockSpec(memory_space=pltpu.MemorySpace.VMEM),
        ],
        out_shape=[
            jax.ShapeDtypeStruct((B, T, D), jnp.float32),
            jax.ShapeDtypeStruct((B, T), jnp.bool_),
        ],
        scratch_shapes=[
            pltpu.VMEM((1, BT, D), jnp.float32),
            pltpu.SemaphoreType.DMA((NSEM,)),
        ],
    )(start_idx, x, mask_token.reshape(1, D))
    return (x_masked, mask)


# speedup vs baseline: 1.0317x; 1.0317x over previous
import jax
import jax.numpy as jnp
from jax import lax
from jax.experimental import pallas as pl
from jax.experimental.pallas import tpu as pltpu

MASK_LEN = 4915
B, T, D = 4, 8192, 1024
BT = 1024
NTB = T // BT
NSEM = 8


def _body(start_ref, x_hbm, mt_ref, out_hbm, mask_ref, tile, sems):
    L = MASK_LEN

    tile[...] = jnp.broadcast_to(mt_ref[...][None], (1, BT, D))
    pos_row = lax.broadcasted_iota(jnp.int32, (1, T), 1)
    for b in range(B):
        s = start_ref[b]
        mask_ref[b : b + 1, :] = (pos_row >= s) & (pos_row < s + L)

    fills = []
    q = 0
    for b in range(B):
        s = start_ref[b]
        sb = s // BT
        eb = (s + L - 1) // BT
        for t in range(1, NTB - 1):
            d = pltpu.make_async_copy(
                tile.at[pl.ds(0, 1), pl.ds(0, BT)],
                out_hbm.at[pl.ds(b, 1), pl.ds(t * BT, BT)],
                sems.at[q % NSEM])
            fills.append(((t > sb) & (t < eb), d))
            q += 1
    for cond, d in fills:
        pl.when(cond)(d.start)

    def blk_of(b):
        s = start_ref[b]
        return s // BT, (s + L - 1) // BT

    def skip_index(b, t):
        sb, eb = blk_of(b)
        interior = (t > sb) & (t < eb)
        return (b, jnp.where(interior, sb, t), 0)

    def copy_body(idx, x_blk, out_blk):
        b, t = idx
        s = start_ref[b]
        sb = s // BT
        eb = (s + L - 1) // BT
        interior = (t > sb) & (t < eb)

        @pl.when(jnp.logical_not(interior))
        def _():
            pos = lax.broadcasted_iota(jnp.int32, (BT, 1), 0) + t * BT
            m = (pos >= s) & (pos < s + L)
            out_blk[0] = jnp.where(m, mt_ref[...], x_blk[0])

    pltpu.emit_pipeline(
        copy_body,
        grid=(B, NTB),
        in_specs=[pl.BlockSpec((1, BT, D), skip_index,
                               pipeline_mode=pl.Buffered(buffer_count=4))],
        out_specs=[pl.BlockSpec((1, BT, D), skip_index)],
        _explicit_indices=True,
    )(x_hbm, out_hbm)

    for cond, d in fills:
        pl.when(cond)(d.wait)


def kernel(x, start_idx, mask_token):
    start_idx = start_idx.astype(jnp.int32)
    x_masked, mask = pl.pallas_call(
        _body,
        in_specs=[
            pl.BlockSpec(memory_space=pltpu.MemorySpace.SMEM),
            pl.BlockSpec(memory_space=pl.ANY),
            pl.BlockSpec(memory_space=pltpu.MemorySpace.VMEM),
        ],
        out_specs=[
            pl.BlockSpec(memory_space=pl.ANY),
            pl.BlockSpec(memory_space=pltpu.MemorySpace.VMEM),
        ],
        out_shape=[
            jax.ShapeDtypeStruct((B, T, D), jnp.float32),
            jax.ShapeDtypeStruct((B, T), jnp.bool_),
        ],
        scratch_shapes=[
            pltpu.VMEM((1, BT, D), jnp.float32),
            pltpu.SemaphoreType.DMA((NSEM,)),
        ],
    )(start_idx, x, mask_token.reshape(1, D))
    return (x_masked, mask)
